# baseline (speedup 1.0000x reference)
import functools

import jax
import jax.numpy as jnp
from jax.experimental import pallas as pl
from jax.experimental.pallas import tpu as pltpu

EPS_NORM = 1e-06
EPS_MEAN = 1e-06

_S = 2048
_C = 6
_L = _S * _C
_BB = 32


def _shift(a, k):
    return jnp.concatenate([a[:, k:], a[:, :k]], axis=1)


def _sig_block(z_ref, out_ref):
    z = z_ref[...]
    f32 = jnp.float32

    s1 = _shift(z, 1)
    s6 = _shift(z, 6)
    s7 = _shift(s6, 1)

    q = z * z
    p = q + _shift(q, 1)
    p6 = _shift(p, 6)

    lane = jax.lax.broadcasted_iota(jnp.int32, (1, _L), 1)
    c = lane % _C

    d = jnp.abs(s6 - z)
    in_range = lane < _C * (_S - 1)
    mvel = (c >= 2) & (c <= 3) & in_range
    macc = (c >= 4) & in_range
    sd_vel = jnp.sum(jnp.where(mvel, d, 0.0), axis=1, keepdims=True)
    sd_acc = jnp.sum(jnp.where(macc, d, 0.0), axis=1, keepdims=True)

    rt = jnp.sqrt(p)
    mspeed = c == 2
    mforce = c == 4
    ss1 = jnp.sum(jnp.where(mspeed, rt, 0.0), axis=1, keepdims=True)
    ss2 = jnp.sum(jnp.where(mspeed, p, 0.0), axis=1, keepdims=True)
    sf1 = jnp.sum(jnp.where(mforce, rt, 0.0), axis=1, keepdims=True)
    sf2 = jnp.sum(jnp.where(mforce, p, 0.0), axis=1, keepdims=True)

    cross = z * s7 - s1 * s6
    norms = jnp.sqrt(p * p6)
    mrange = (c == 0) & (lane >= _C) & (lane <= _C * (_S - 2))
    good = mrange & (norms > EPS_NORM)
    curv = jnp.where(good, jnp.abs(cross) / jnp.where(good, norms, 1.0), 0.0)
    scurv = jnp.sum(curv, axis=1, keepdims=True)
    scnt = jnp.sum(jnp.where(good, f32(1.0), f32(0.0)), axis=1, keepdims=True)

    pc = jnp.where(scnt > 0, scurv / jnp.maximum(scnt, 1.0), 0.0)
    vs = 1.0 / (1.0 + sd_vel * (1.0 / (2 * (_S - 1))))
    aj = sd_acc * (1.0 / (2 * (_S - 1)))

    mean_s = ss1 * (1.0 / _S)
    var_s = jnp.maximum(ss2 * (1.0 / _S) - mean_s * mean_s, 0.0)
    mr = jnp.sqrt(var_s) / (mean_s + EPS_MEAN)

    mean_f = sf1 * (1.0 / _S)
    var_f = jnp.maximum(sf2 * (1.0 / _S) - mean_f * mean_f, 0.0)
    fm = jnp.sqrt(var_f) / (mean_f + EPS_MEAN)

    out_ref[...] = jnp.concatenate([pc, vs, aj, mr, fm], axis=1)


@jax.jit
def kernel(trajectories):
    b = trajectories.shape[0]
    z = trajectories.reshape(b, _L)
    grid = (b // _BB,)
    return pl.pallas_call(
        _sig_block,
        grid=grid,
        in_specs=[pl.BlockSpec((_BB, _L), lambda i: (i, 0))],
        out_specs=pl.BlockSpec((_BB, 5), lambda i: (i, 0)),
        out_shape=jax.ShapeDtypeStruct((b, 5), jnp.float32),
        compiler_params=pltpu.CompilerParams(
            dimension_semantics=("parallel",),
        ),
    )(z)
